# baseline (speedup 1.0000x reference)
import functools

import jax
import jax.numpy as jnp
from jax import lax
from jax.experimental import pallas as pl
from jax.experimental.pallas import tpu as pltpu
from jax.experimental.pallas import tpu_sc as plsc

HZ_ = 100
BS_ = 16
SLEN_ = 30000
CH_ = 12
NSEG_ = SLEN_ // HZ_
NCH_ = CH_ * HZ_
NVREG_ = NCH_ // 16
WIN_ = 256
SK_ = 129
BSTR_ = CH_ * SK_
NCHUNK_ = 8
SROUNDS_ = 10


def _sc_body(x_hbm, sl_hbm, pidx_hbm, out_hbm, osl_hbm,
             buf, ts_flat, ts_out, pidx_v, sl_v, rsem, wsem):
    info = plsc.get_sparse_core_info()
    nworkers = info.num_cores * info.num_subcores
    wid = lax.axis_index("s") * info.num_cores + lax.axis_index("c")
    iota = lax.iota(jnp.int32, 16)

    pltpu.sync_copy(pidx_hbm, pidx_v)

    @pl.when(wid == 0)
    def _():
        pltpu.sync_copy(sl_hbm, sl_v)
        sl_v[...] = lax.div(sl_v[...], iota * 0 + HZ_)
        pltpu.sync_copy(sl_v, osl_hbm)

    def _a0(s):
        return pl.multiple_of(((s * HZ_) // 128) * 128, 128)

    def _read_copy(s):
        return pltpu.make_async_copy(
            x_hbm.at[:, :, pl.ds(_a0(s), WIN_)], buf, rsem
        )

    _read_copy(wid).start()

    def _seg(k, carry):
        s = wid + nworkers * k

        @pl.when(s < NSEG_)
        def _():
            t0 = s * HZ_
            off = t0 - _a0(s)
            c0 = pl.multiple_of((off // 16) * 16, 16)
            off2 = off - c0
            _read_copy(s).wait()

            def _cstep(c, carry2):
                coff = c * SK_
                for b in range(BS_):
                    gs = [
                        buf[c, b, pl.ds(c0 + 16 * m, 16)] for m in range(NCHUNK_)
                    ]
                    for m in range(NCHUNK_):
                        ts_flat[pl.ds(b * BSTR_ + coff + 16 * m, 16)] = gs[m]
                return carry2

            lax.fori_loop(0, CH_, _cstep, 0, unroll=2)

            s_nxt = s + nworkers

            @pl.when(s_nxt < NSEG_)
            def _():
                _read_copy(s_nxt).start()

            @pl.when(s >= nworkers)
            def _():
                pltpu.make_async_copy(ts_out, out_hbm.at[s - nworkers], wsem).wait()

            off2v = iota * 0 + off2

            def _vstep(v, carry2):
                base = pidx_v[pl.ds(16 * v, 16)] + off2v
                gs = [
                    plsc.load_gather(ts_flat, [base + (b * BSTR_)])
                    for b in range(BS_)
                ]
                for b in range(BS_):
                    ts_out[b, pl.ds(16 * v, 16)] = gs[b]
                return carry2

            lax.fori_loop(0, NVREG_, _vstep, 0, unroll=3)
            pltpu.make_async_copy(ts_out, out_hbm.at[s], wsem).start()

        return carry

    lax.fori_loop(0, SROUNDS_, _seg, 0)

    s_last = wid + nworkers * ((NSEG_ - 1 - wid) // nworkers)
    pltpu.make_async_copy(ts_out, out_hbm.at[s_last], wsem).wait()


def kernel(x, seq_lens):
    xt = jnp.transpose(x, (2, 0, 1))
    sl = seq_lens.astype(jnp.int32)
    didx = jnp.arange(NCH_, dtype=jnp.int32)
    pidx = (didx % CH_) * SK_ + didx // CH_
    mesh = plsc.VectorSubcoreMesh(core_axis_name="c", subcore_axis_name="s")
    sc = functools.partial(
        pl.kernel,
        out_type=[
            jax.ShapeDtypeStruct((NSEG_, BS_, NCH_), x.dtype),
            jax.ShapeDtypeStruct((BS_,), jnp.int32),
        ],
        scratch_types=[
            pltpu.VMEM((CH_, BS_, WIN_), jnp.float32),
            pltpu.VMEM((BS_ * BSTR_,), jnp.float32),
            pltpu.VMEM((BS_, NCH_), jnp.float32),
            pltpu.VMEM((NCH_,), jnp.int32),
            pltpu.VMEM((BS_,), jnp.int32),
            pltpu.SemaphoreType.DMA,
            pltpu.SemaphoreType.DMA,
        ],
        mesh=mesh,
        compiler_params=pltpu.CompilerParams(needs_layout_passes=False),
    )(_sc_body)
    out_t, osl = sc(xt, sl, pidx)
    return jnp.transpose(out_t, (1, 0, 2)), osl.astype(seq_lens.dtype)


# speedup vs baseline: 1.1579x; 1.1579x over previous
import functools

import jax
import jax.numpy as jnp
from jax import lax
from jax.experimental import pallas as pl
from jax.experimental.pallas import tpu as pltpu
from jax.experimental.pallas import tpu_sc as plsc

HZ_ = 100
BS_ = 16
SLEN_ = 30000
CH_ = 12
CHH_ = CH_ // 2
NSEG_ = SLEN_ // HZ_
NCH_ = CH_ * HZ_
NVREG_ = NCH_ // 16
WIN_ = 256
SK_ = 129
BSTR_ = CH_ * SK_
NCHUNK_ = 8
SROUNDS_ = 10


def _sc_body(x_hbm, sl_hbm, pidx_hbm, out_hbm, osl_hbm,
             buf, ts_flat, ts_out, pidx_v, sl_v, rsem_a, rsem_b, wsem):
    info = plsc.get_sparse_core_info()
    nworkers = info.num_cores * info.num_subcores
    wid = lax.axis_index("s") * info.num_cores + lax.axis_index("c")
    iota = lax.iota(jnp.int32, 16)

    pltpu.sync_copy(pidx_hbm, pidx_v)

    @pl.when(wid == 0)
    def _():
        pltpu.sync_copy(sl_hbm, sl_v)
        sl_v[...] = lax.div(sl_v[...], iota * 0 + HZ_)
        pltpu.sync_copy(sl_v, osl_hbm)

    def _a0(s):
        return pl.multiple_of(((s * HZ_) // 128) * 128, 128)

    def _read_copy(s, half, sem):
        return pltpu.make_async_copy(
            x_hbm.at[pl.ds(half * CHH_, CHH_), :, pl.ds(_a0(s), WIN_)],
            buf.at[pl.ds(half * CHH_, CHH_)],
            sem,
        )

    _read_copy(wid, 0, rsem_a).start()
    _read_copy(wid, 1, rsem_b).start()

    def _stage2(c0, clo):
        def _cstep(c, carry2):
            coff = c * SK_
            for b in range(BS_):
                gs = [
                    buf[c, b, pl.ds(c0 + 16 * m, 16)] for m in range(NCHUNK_)
                ]
                for m in range(NCHUNK_):
                    ts_flat[pl.ds(b * BSTR_ + coff + 16 * m, 16)] = gs[m]
            return carry2

        lax.fori_loop(clo, clo + CHH_, _cstep, 0)

    def _seg(k, carry):
        s = wid + nworkers * k

        @pl.when(s < NSEG_)
        def _():
            t0 = s * HZ_
            off = t0 - _a0(s)
            c0 = pl.multiple_of((off // 16) * 16, 16)
            off2 = off - c0
            s_nxt = s + nworkers

            _read_copy(s, 0, rsem_a).wait()
            _stage2(c0, 0)

            @pl.when(s_nxt < NSEG_)
            def _():
                _read_copy(s_nxt, 0, rsem_a).start()

            _read_copy(s, 1, rsem_b).wait()
            _stage2(c0, CHH_)

            @pl.when(s_nxt < NSEG_)
            def _():
                _read_copy(s_nxt, 1, rsem_b).start()

            @pl.when(s >= nworkers)
            def _():
                pltpu.make_async_copy(ts_out, out_hbm.at[s - nworkers], wsem).wait()

            off2v = iota * 0 + off2

            def _vstep(v, carry2):
                base = pidx_v[pl.ds(16 * v, 16)] + off2v
                gs = [
                    plsc.load_gather(ts_flat, [base + (b * BSTR_)])
                    for b in range(BS_)
                ]
                for b in range(BS_):
                    ts_out[b, pl.ds(16 * v, 16)] = gs[b]
                return carry2

            lax.fori_loop(0, NVREG_, _vstep, 0)
            pltpu.make_async_copy(ts_out, out_hbm.at[s], wsem).start()

        return carry

    lax.fori_loop(0, SROUNDS_, _seg, 0)

    s_last = wid + nworkers * ((NSEG_ - 1 - wid) // nworkers)
    pltpu.make_async_copy(ts_out, out_hbm.at[s_last], wsem).wait()


def kernel(x, seq_lens):
    xt = jnp.transpose(x, (2, 0, 1))
    sl = seq_lens.astype(jnp.int32)
    didx = jnp.arange(NCH_, dtype=jnp.int32)
    pidx = (didx % CH_) * SK_ + didx // CH_
    mesh = plsc.VectorSubcoreMesh(core_axis_name="c", subcore_axis_name="s")
    sc = functools.partial(
        pl.kernel,
        out_type=[
            jax.ShapeDtypeStruct((NSEG_, BS_, NCH_), x.dtype),
            jax.ShapeDtypeStruct((BS_,), jnp.int32),
        ],
        scratch_types=[
            pltpu.VMEM((CH_, BS_, WIN_), jnp.float32),
            pltpu.VMEM((BS_ * BSTR_,), jnp.float32),
            pltpu.VMEM((BS_, NCH_), jnp.float32),
            pltpu.VMEM((NCH_,), jnp.int32),
            pltpu.VMEM((BS_,), jnp.int32),
            pltpu.SemaphoreType.DMA,
            pltpu.SemaphoreType.DMA,
            pltpu.SemaphoreType.DMA,
        ],
        mesh=mesh,
        compiler_params=pltpu.CompilerParams(needs_layout_passes=False),
    )(_sc_body)
    out_t, osl = sc(xt, sl, pidx)
    return jnp.transpose(out_t, (1, 0, 2)), osl.astype(seq_lens.dtype)
